# baseline (speedup 1.0000x reference)
import jax
import jax.numpy as jnp
from jax import lax
from jax.experimental import pallas as pl
from jax.experimental.pallas import tpu as pltpu
from jax.experimental.pallas import tpu_sc as plsc

N = 10000
E = 160000
D_IN = 256
D_H = 512
N_HOPS = 3
N_MLP = N_HOPS + 1

NC, NS, LANES = 2, 16, 16
DHALF = D_IN // 2
ROWS_PER_TILE = 640
N_SP = NS * ROWS_PER_TILE
CHUNK = 128
CHUNKS_PER_TILE = 80
E_PAD = NS * CHUNKS_PER_TILE * CHUNK
NORM_CH = 128
ROW_BLK = 256
N_BLKS = N_SP // ROW_BLK


def _sc_body(xcat_hbm, src_hbm, dst_hbm, o1_hbm, o2_hbm, o3_hbm,
             acc_sp, deg_sp, sem_a, sem_b):
    c = lax.axis_index("c")
    s = lax.axis_index("s")
    c_off = c * N_SP
    tile_row0 = s * ROWS_PER_TILE
    e_base = s * CHUNKS_PER_TILE * CHUNK

    def scoped(src_a, src_b, dst2d, rows_a, rows_b, ones_v, dinv):
        buf = rows_a
        for t in range(ROWS_PER_TILE // LANES):
            dinv[pl.ds(t * LANES, LANES)] = jnp.zeros((LANES,), jnp.float32)
        for t in range(CHUNK // LANES):
            ones_v[pl.ds(t * LANES, LANES)] = jnp.ones((LANES,), jnp.float32)
        pltpu.sync_copy(dinv, deg_sp.at[pl.ds(tile_row0, ROWS_PER_TILE)])

        pltpu.sync_copy(dst_hbm.at[pl.ds(s * CHUNKS_PER_TILE,
                                         CHUNKS_PER_TILE)], dst2d)
        plsc.subcore_barrier()

        def load_src(j, sbuf):
            pltpu.sync_copy(src_hbm.at[pl.ds(e_base + j * CHUNK, CHUNK)], sbuf)
            for t in range(CHUNK // LANES):
                sl = pl.ds(t * LANES, LANES)
                sbuf[sl] = sbuf[sl] + c_off

        outs = [o1_hbm, o2_hbm, o3_hbm]
        h_in = xcat_hbm
        for hop in range(N_HOPS):
            for ch in range(ROWS_PER_TILE // NORM_CH):
                r0 = tile_row0 + ch * NORM_CH
                pltpu.sync_copy(h_in.at[pl.ds(c_off + r0, NORM_CH)], buf)
                pltpu.sync_copy(buf, acc_sp.at[pl.ds(r0, NORM_CH)])
            plsc.subcore_barrier()

            def start_g(sbuf, rbuf, sem):
                pltpu.async_copy(h_in.at[sbuf], rbuf, sem)

            def wait_g(rbuf, sem):
                pltpu.make_async_copy(h_in.at[src_a], rbuf, sem).wait()

            def scat(j, rbuf):
                pltpu.sync_copy(rbuf, acc_sp.at[dst2d.at[j]], add=True)
                if hop == 0:
                    pltpu.sync_copy(ones_v, deg_sp.at[dst2d.at[j]], add=True)

            load_src(0, src_a)
            start_g(src_a, rows_a, sem_a)
            load_src(1, src_b)
            start_g(src_b, rows_b, sem_b)

            def pipe(j2, carry):
                a = 2 * j2
                wait_g(rows_a, sem_a)
                scat(a, rows_a)
                load_src(a + 2, src_a)
                start_g(src_a, rows_a, sem_a)
                wait_g(rows_b, sem_b)
                scat(a + 1, rows_b)
                load_src(a + 3, src_b)
                start_g(src_b, rows_b, sem_b)
                return carry
            lax.fori_loop(0, (CHUNKS_PER_TILE - 2) // 2, pipe, 0)
            wait_g(rows_a, sem_a)
            scat(CHUNKS_PER_TILE - 2, rows_a)
            wait_g(rows_b, sem_b)
            scat(CHUNKS_PER_TILE - 1, rows_b)
            plsc.subcore_barrier()

            if hop == 0:
                pltpu.sync_copy(deg_sp.at[pl.ds(tile_row0, ROWS_PER_TILE)],
                                dinv)
                for t in range(ROWS_PER_TILE // LANES):
                    dv = dinv[pl.ds(t * LANES, LANES)]
                    dinv[pl.ds(t * LANES, LANES)] = 1.0 / (1.0 + dv)

            out = outs[hop]
            for ch in range(ROWS_PER_TILE // NORM_CH):
                row0 = tile_row0 + ch * NORM_CH
                pltpu.sync_copy(acc_sp.at[pl.ds(row0, NORM_CH)], buf)

                def norm_grp(grp, carry):
                    dvv = dinv[pl.ds(ch * NORM_CH + grp * LANES, LANES)]
                    for r in range(LANES):
                        dv = dvv[r]
                        row = grp * LANES + r
                        for t in range(DHALF // LANES):
                            sl = pl.ds(t * LANES, LANES)
                            buf[row, sl] = buf[row, sl] * dv
                    return carry
                lax.fori_loop(0, NORM_CH // LANES, norm_grp, 0)
                pltpu.sync_copy(buf, out.at[pl.ds(c_off + row0, NORM_CH)])
            plsc.subcore_barrier()
            h_in = out

    pl.run_scoped(scoped,
                  pltpu.VMEM((CHUNK,), jnp.int32),
                  pltpu.VMEM((CHUNK,), jnp.int32),
                  pltpu.VMEM((CHUNKS_PER_TILE, CHUNK), jnp.int32),
                  pltpu.VMEM((CHUNK, DHALF), jnp.float32),
                  pltpu.VMEM((CHUNK, DHALF), jnp.float32),
                  pltpu.VMEM((CHUNK,), jnp.float32),
                  pltpu.VMEM((ROWS_PER_TILE,), jnp.float32))


def _sc_propagate(xcat, src_p, dst_p):
    mesh = plsc.VectorSubcoreMesh(core_axis_name="c", subcore_axis_name="s",
                                  num_cores=NC, num_subcores=NS)
    k = pl.kernel(
        _sc_body,
        out_type=[jax.ShapeDtypeStruct((NC * N_SP, DHALF), jnp.float32)] * 3,
        mesh=mesh,
        scratch_types=[
            pltpu.VMEM_SHARED((N_SP, DHALF), jnp.float32),
            pltpu.VMEM_SHARED((N_SP,), jnp.float32),
            pltpu.SemaphoreType.DMA,
            pltpu.SemaphoreType.DMA,
        ],
    )
    return k(xcat, src_p, dst_p)


def _ln(z, gamma, beta):
    mu = jnp.mean(z, axis=1, keepdims=True)
    d = z - mu
    var = jnp.mean(d * d, axis=1, keepdims=True)
    return d * lax.rsqrt(var + 1e-5) * gamma + beta


def _mlp_body(x_ref, a1, b1, a2, b2, a3, b3, W_ref, b_ref, g_ref, be_ref,
              Wf_ref, bf_ref, gf_ref, bef_ref, o_ref):
    halves = [(a1, b1), (a2, b2), (a3, b3)]
    acc = None
    for k in range(N_MLP):
        if k == 0:
            z = jnp.dot(x_ref[...], W_ref[0],
                        preferred_element_type=jnp.float32)
        else:
            ha, hb = halves[k - 1]
            z = jnp.dot(ha[...], W_ref[k, :DHALF, :],
                        preferred_element_type=jnp.float32)
            z = z + jnp.dot(hb[...], W_ref[k, DHALF:, :],
                            preferred_element_type=jnp.float32)
        z = z + b_ref[k]
        z = jnp.maximum(z, 0.0)
        z = _ln(z, g_ref[k], be_ref[k])
        acc = z if acc is None else acc + z
    out = jnp.dot(acc, Wf_ref[...], preferred_element_type=jnp.float32)
    out = jnp.maximum(out + bf_ref[0], 0.0)
    o_ref[...] = _ln(out, gf_ref[0], bef_ref[0])


def _tc_mlps(xp, o1, o2, o3, W, b, g, be, Wf, bf, gf, bef):
    row_spec = pl.BlockSpec((ROW_BLK, D_IN), lambda i: (i, 0))
    half_a = pl.BlockSpec((ROW_BLK, DHALF), lambda i: (i, 0))
    half_b = pl.BlockSpec((ROW_BLK, DHALF), lambda i: (i + N_BLKS, 0))
    full = lambda *shape: pl.BlockSpec(shape, lambda i: (0,) * len(shape))
    return pl.pallas_call(
        _mlp_body,
        grid=(N_BLKS,),
        in_specs=[
            row_spec,
            half_a, half_b, half_a, half_b, half_a, half_b,
            full(N_MLP, D_IN, D_H),
            full(N_MLP, D_H), full(N_MLP, D_H), full(N_MLP, D_H),
            full(D_H, D_H),
            full(1, D_H), full(1, D_H), full(1, D_H),
        ],
        out_specs=pl.BlockSpec((ROW_BLK, D_H), lambda i: (i, 0)),
        out_shape=jax.ShapeDtypeStruct((N_SP, D_H), jnp.float32),
    )(xp, o1, o1, o2, o2, o3, o3, W, b, g, be, Wf,
      bf.reshape(1, D_H), gf.reshape(1, D_H), bef.reshape(1, D_H))


@jax.jit
def kernel(x, edge_index, W, b, g, be, Wf, bf, gf, bef):
    src, dst = edge_index[0], edge_index[1]
    pad_e = E_PAD - E

    src_p = jnp.concatenate([src, jnp.zeros((pad_e,), jnp.int32)])
    dst_p = jnp.concatenate([dst, jnp.full((pad_e,), N, jnp.int32)]).reshape(
        NS * CHUNKS_PER_TILE, CHUNK)

    xp = jnp.pad(x, ((0, N_SP - N), (0, 0)))
    xcat = jnp.concatenate([xp[:, :DHALF], xp[:, DHALF:]])

    o1, o2, o3 = _sc_propagate(xcat, src_p, dst_p)
    out = _tc_mlps(xp, o1, o2, o3, W, b, g, be, Wf, bf, gf, bef)
    return out[:N]


# speedup vs baseline: 1.0435x; 1.0435x over previous
import jax
import jax.numpy as jnp
from jax import lax
from jax.experimental import pallas as pl
from jax.experimental.pallas import tpu as pltpu
from jax.experimental.pallas import tpu_sc as plsc

N = 10000
E = 160000
D_IN = 256
D_H = 512
N_HOPS = 3
N_MLP = N_HOPS + 1

NC, NS, LANES = 2, 16, 16
DHALF = D_IN // 2
ROWS_PER_TILE = 640
N_SP = NS * ROWS_PER_TILE
CHUNK = 128
CHUNKS_PER_TILE = 80
E_PAD = NS * CHUNKS_PER_TILE * CHUNK
NORM_CH = 128
ROW_BLK = 256
N_BLKS = N_SP // ROW_BLK


def _sc_body(xcat_hbm, src_hbm, dst_hbm, o1_hbm, o2_hbm, o3_hbm,
             acc_sp, deg_sp, sem_a, sem_b):
    c = lax.axis_index("c")
    s = lax.axis_index("s")
    c_off = c * N_SP
    tile_row0 = s * ROWS_PER_TILE
    e_base = s * CHUNKS_PER_TILE * CHUNK

    def scoped(src2d, dst2d, rows_a, rows_b, ones_v, dinv):
        buf = rows_a
        for t in range(ROWS_PER_TILE // LANES):
            dinv[pl.ds(t * LANES, LANES)] = jnp.zeros((LANES,), jnp.float32)
        for t in range(CHUNK // LANES):
            ones_v[pl.ds(t * LANES, LANES)] = jnp.ones((LANES,), jnp.float32)
        pltpu.sync_copy(dinv, deg_sp.at[pl.ds(tile_row0, ROWS_PER_TILE)])

        pltpu.sync_copy(dst_hbm.at[pl.ds(s * CHUNKS_PER_TILE,
                                         CHUNKS_PER_TILE)], dst2d)
        plsc.subcore_barrier()

        outs = [o1_hbm, o2_hbm, o3_hbm]
        h_in = xcat_hbm
        for hop in range(N_HOPS):
            for ch in range(ROWS_PER_TILE // NORM_CH):
                r0 = tile_row0 + ch * NORM_CH
                pltpu.sync_copy(h_in.at[pl.ds(c_off + r0, NORM_CH)], buf)
                pltpu.sync_copy(buf, acc_sp.at[pl.ds(r0, NORM_CH)])
            plsc.subcore_barrier()

            def start_g(j, rbuf, sem):
                pltpu.async_copy(h_in.at[src2d.at[j]], rbuf, sem)

            def wait_g(rbuf, sem):
                pltpu.make_async_copy(h_in.at[src2d.at[0]], rbuf, sem).wait()

            def scat(j, rbuf):
                pltpu.sync_copy(rbuf, acc_sp.at[dst2d.at[j]], add=True)
                if hop == 0:
                    pltpu.sync_copy(ones_v, deg_sp.at[dst2d.at[j]], add=True)

            for j0, nb in ((0, 32), (32, 32), (64, 16)):
                pltpu.sync_copy(
                    src_hbm.at[pl.ds(s * CHUNKS_PER_TILE + j0, nb)],
                    src2d.at[pl.ds(0, nb)])

                def add_off(j, carry):
                    for t in range(CHUNK // LANES):
                        sl = pl.ds(t * LANES, LANES)
                        src2d[j, sl] = src2d[j, sl] + c_off
                    return carry
                lax.fori_loop(0, nb, add_off, 0)

                start_g(0, rows_a, sem_a)
                start_g(1, rows_b, sem_b)

                def pipe(j2, carry):
                    a = 2 * j2
                    wait_g(rows_a, sem_a)
                    scat(j0 + a, rows_a)
                    start_g(a + 2, rows_a, sem_a)
                    wait_g(rows_b, sem_b)
                    scat(j0 + a + 1, rows_b)
                    start_g(a + 3, rows_b, sem_b)
                    return carry
                lax.fori_loop(0, (nb - 2) // 2, pipe, 0)
                wait_g(rows_a, sem_a)
                scat(j0 + nb - 2, rows_a)
                wait_g(rows_b, sem_b)
                scat(j0 + nb - 1, rows_b)
            plsc.subcore_barrier()

            if hop == 0:
                pltpu.sync_copy(deg_sp.at[pl.ds(tile_row0, ROWS_PER_TILE)],
                                dinv)
                for t in range(ROWS_PER_TILE // LANES):
                    dv = dinv[pl.ds(t * LANES, LANES)]
                    dinv[pl.ds(t * LANES, LANES)] = 1.0 / (1.0 + dv)

            out = outs[hop]
            for ch in range(ROWS_PER_TILE // NORM_CH):
                row0 = tile_row0 + ch * NORM_CH
                pltpu.sync_copy(acc_sp.at[pl.ds(row0, NORM_CH)], buf)

                def norm_grp(grp, carry):
                    dvv = dinv[pl.ds(ch * NORM_CH + grp * LANES, LANES)]
                    for r in range(LANES):
                        dv = dvv[r]
                        row = grp * LANES + r
                        for t in range(DHALF // LANES):
                            sl = pl.ds(t * LANES, LANES)
                            buf[row, sl] = buf[row, sl] * dv
                    return carry
                lax.fori_loop(0, NORM_CH // LANES, norm_grp, 0)
                pltpu.sync_copy(buf, out.at[pl.ds(c_off + row0, NORM_CH)])
            plsc.subcore_barrier()
            h_in = out

    pl.run_scoped(scoped,
                  pltpu.VMEM((32, CHUNK), jnp.int32),
                  pltpu.VMEM((CHUNKS_PER_TILE, CHUNK), jnp.int32),
                  pltpu.VMEM((CHUNK, DHALF), jnp.float32),
                  pltpu.VMEM((CHUNK, DHALF), jnp.float32),
                  pltpu.VMEM((CHUNK,), jnp.float32),
                  pltpu.VMEM((ROWS_PER_TILE,), jnp.float32))


def _sc_propagate(xcat, src_p, dst_p):
    mesh = plsc.VectorSubcoreMesh(core_axis_name="c", subcore_axis_name="s",
                                  num_cores=NC, num_subcores=NS)
    k = pl.kernel(
        _sc_body,
        out_type=[jax.ShapeDtypeStruct((NC * N_SP, DHALF), jnp.float32)] * 3,
        mesh=mesh,
        scratch_types=[
            pltpu.VMEM_SHARED((N_SP, DHALF), jnp.float32),
            pltpu.VMEM_SHARED((N_SP,), jnp.float32),
            pltpu.SemaphoreType.DMA,
            pltpu.SemaphoreType.DMA,
        ],
    )
    return k(xcat, src_p, dst_p)


def _ln(z, gamma, beta):
    mu = jnp.mean(z, axis=1, keepdims=True)
    d = z - mu
    var = jnp.mean(d * d, axis=1, keepdims=True)
    return d * lax.rsqrt(var + 1e-5) * gamma + beta


def _mlp_body(x_ref, a1, b1, a2, b2, a3, b3, W_ref, b_ref, g_ref, be_ref,
              Wf_ref, bf_ref, gf_ref, bef_ref, o_ref):
    halves = [(a1, b1), (a2, b2), (a3, b3)]
    acc = None
    for k in range(N_MLP):
        if k == 0:
            z = jnp.dot(x_ref[...], W_ref[0],
                        preferred_element_type=jnp.float32)
        else:
            ha, hb = halves[k - 1]
            z = jnp.dot(ha[...], W_ref[k, :DHALF, :],
                        preferred_element_type=jnp.float32)
            z = z + jnp.dot(hb[...], W_ref[k, DHALF:, :],
                            preferred_element_type=jnp.float32)
        z = z + b_ref[k]
        z = jnp.maximum(z, 0.0)
        z = _ln(z, g_ref[k], be_ref[k])
        acc = z if acc is None else acc + z
    out = jnp.dot(acc, Wf_ref[...], preferred_element_type=jnp.float32)
    out = jnp.maximum(out + bf_ref[0], 0.0)
    o_ref[...] = _ln(out, gf_ref[0], bef_ref[0])


def _tc_mlps(xp, o1, o2, o3, W, b, g, be, Wf, bf, gf, bef):
    row_spec = pl.BlockSpec((ROW_BLK, D_IN), lambda i: (i, 0))
    half_a = pl.BlockSpec((ROW_BLK, DHALF), lambda i: (i, 0))
    half_b = pl.BlockSpec((ROW_BLK, DHALF), lambda i: (i + N_BLKS, 0))
    full = lambda *shape: pl.BlockSpec(shape, lambda i: (0,) * len(shape))
    return pl.pallas_call(
        _mlp_body,
        grid=(N_BLKS,),
        in_specs=[
            row_spec,
            half_a, half_b, half_a, half_b, half_a, half_b,
            full(N_MLP, D_IN, D_H),
            full(N_MLP, D_H), full(N_MLP, D_H), full(N_MLP, D_H),
            full(D_H, D_H),
            full(1, D_H), full(1, D_H), full(1, D_H),
        ],
        out_specs=pl.BlockSpec((ROW_BLK, D_H), lambda i: (i, 0)),
        out_shape=jax.ShapeDtypeStruct((N_SP, D_H), jnp.float32),
    )(xp, o1, o1, o2, o2, o3, o3, W, b, g, be, Wf,
      bf.reshape(1, D_H), gf.reshape(1, D_H), bef.reshape(1, D_H))


@jax.jit
def kernel(x, edge_index, W, b, g, be, Wf, bf, gf, bef):
    src, dst = edge_index[0], edge_index[1]
    pad_e = E_PAD - E

    src_p = jnp.concatenate([src, jnp.zeros((pad_e,), jnp.int32)]).reshape(
        NS * CHUNKS_PER_TILE, CHUNK)
    dst_p = jnp.concatenate([dst, jnp.full((pad_e,), N, jnp.int32)]).reshape(
        NS * CHUNKS_PER_TILE, CHUNK)

    xp = jnp.pad(x, ((0, N_SP - N), (0, 0)))
    xcat = jnp.concatenate([xp[:, :DHALF], xp[:, DHALF:]])

    o1, o2, o3 = _sc_propagate(xcat, src_p, dst_p)
    out = _tc_mlps(xp, o1, o2, o3, W, b, g, be, Wf, bf, gf, bef)
    return out[:N]


# speedup vs baseline: 1.1465x; 1.0987x over previous
import jax
import jax.numpy as jnp
from jax import lax
from jax.experimental import pallas as pl
from jax.experimental.pallas import tpu as pltpu
from jax.experimental.pallas import tpu_sc as plsc

N = 10000
E = 160000
D_IN = 256
D_H = 512
N_HOPS = 3
N_MLP = N_HOPS + 1

NC, NS, LANES = 2, 16, 16
DHALF = D_IN // 2
ROWS_PER_TILE = 640
N_SP = NS * ROWS_PER_TILE
CHUNK = 128
CHUNKS_PER_TILE = 79
TILE_STRIDE = 80
E_PAD = NS * CHUNKS_PER_TILE * CHUNK
NORM_CH = 128
ROW_BLK = 256
N_BLKS = N_SP // ROW_BLK


def _sc_body(xcat_hbm, src_hbm, dst_hbm, o1_hbm, o2_hbm, o3_hbm,
             acc_sp, deg_sp, sem_a, sem_b):
    c = lax.axis_index("c")
    s = lax.axis_index("s")
    c_off = c * N_SP
    tile_row0 = s * ROWS_PER_TILE

    def scoped(src2d, dst2d, rows_a, ones_v, dinv):
        buf = rows_a
        for t in range(ROWS_PER_TILE // LANES):
            dinv[pl.ds(t * LANES, LANES)] = jnp.zeros((LANES,), jnp.float32)
        for t in range(CHUNK // LANES):
            ones_v[pl.ds(t * LANES, LANES)] = jnp.ones((LANES,), jnp.float32)
        pltpu.sync_copy(dinv, deg_sp.at[pl.ds(tile_row0, ROWS_PER_TILE)])

        pltpu.sync_copy(src_hbm.at[pl.ds(s * TILE_STRIDE, TILE_STRIDE)], src2d)
        pltpu.sync_copy(dst_hbm.at[pl.ds(s * TILE_STRIDE, TILE_STRIDE)], dst2d)

        def add_off(j, carry):
            for t in range(CHUNK // LANES):
                sl = pl.ds(t * LANES, LANES)
                src2d[j, sl] = src2d[j, sl] + c_off
            return carry
        lax.fori_loop(0, CHUNKS_PER_TILE, add_off, 0)
        plsc.subcore_barrier()

        outs = [o1_hbm, o2_hbm, o3_hbm]
        h_in = xcat_hbm
        for hop in range(N_HOPS):
            for ch in range(ROWS_PER_TILE // NORM_CH):
                r0 = tile_row0 + ch * NORM_CH
                pltpu.sync_copy(h_in.at[pl.ds(c_off + r0, NORM_CH)], buf)
                pltpu.sync_copy(buf, acc_sp.at[pl.ds(r0, NORM_CH)])
            plsc.subcore_barrier()

            def chunk_body(j, carry):
                pltpu.async_copy(h_in.at[src2d.at[j]], rows_a, sem_a).wait()
                pltpu.sync_copy(rows_a, acc_sp.at[dst2d.at[j]], add=True)
                if hop == 0:
                    pltpu.sync_copy(ones_v, deg_sp.at[dst2d.at[j]], add=True)
                return carry
            lax.fori_loop(0, CHUNKS_PER_TILE, chunk_body, 0, unroll=2)
            plsc.subcore_barrier()

            if hop == 0:
                pltpu.sync_copy(deg_sp.at[pl.ds(tile_row0, ROWS_PER_TILE)],
                                dinv)
                for t in range(ROWS_PER_TILE // LANES):
                    dv = dinv[pl.ds(t * LANES, LANES)]
                    dinv[pl.ds(t * LANES, LANES)] = 1.0 / (1.0 + dv)

            out = outs[hop]
            for ch in range(ROWS_PER_TILE // NORM_CH):
                row0 = tile_row0 + ch * NORM_CH
                pltpu.sync_copy(acc_sp.at[pl.ds(row0, NORM_CH)], buf)

                def norm_grp(grp, carry):
                    dvv = dinv[pl.ds(ch * NORM_CH + grp * LANES, LANES)]
                    for r in range(LANES):
                        dv = dvv[r]
                        row = grp * LANES + r
                        for t in range(DHALF // LANES):
                            sl = pl.ds(t * LANES, LANES)
                            buf[row, sl] = buf[row, sl] * dv
                    return carry
                lax.fori_loop(0, NORM_CH // LANES, norm_grp, 0)
                pltpu.sync_copy(buf, out.at[pl.ds(c_off + row0, NORM_CH)])
            plsc.subcore_barrier()
            h_in = out

    pl.run_scoped(scoped,
                  pltpu.VMEM((TILE_STRIDE, CHUNK), jnp.int32),
                  pltpu.VMEM((TILE_STRIDE, CHUNK), jnp.int32),
                  pltpu.VMEM((CHUNK, DHALF), jnp.float32),
                  pltpu.VMEM((CHUNK,), jnp.float32),
                  pltpu.VMEM((ROWS_PER_TILE,), jnp.float32))


def _sc_propagate(xcat, src_p, dst_p):
    mesh = plsc.VectorSubcoreMesh(core_axis_name="c", subcore_axis_name="s",
                                  num_cores=NC, num_subcores=NS)
    k = pl.kernel(
        _sc_body,
        out_type=[jax.ShapeDtypeStruct((NC * N_SP, DHALF), jnp.float32)] * 3,
        mesh=mesh,
        scratch_types=[
            pltpu.VMEM_SHARED((N_SP, DHALF), jnp.float32),
            pltpu.VMEM_SHARED((N_SP,), jnp.float32),
            pltpu.SemaphoreType.DMA,
            pltpu.SemaphoreType.DMA,
        ],
    )
    return k(xcat, src_p, dst_p)


def _ln(z, gamma, beta):
    mu = jnp.mean(z, axis=1, keepdims=True)
    d = z - mu
    var = jnp.mean(d * d, axis=1, keepdims=True)
    return d * lax.rsqrt(var + 1e-5) * gamma + beta


def _mlp_body(x_ref, a1, b1, a2, b2, a3, b3, W_ref, b_ref, g_ref, be_ref,
              Wf_ref, bf_ref, gf_ref, bef_ref, o_ref):
    halves = [(a1, b1), (a2, b2), (a3, b3)]
    acc = None
    for k in range(N_MLP):
        if k == 0:
            z = jnp.dot(x_ref[...], W_ref[0],
                        preferred_element_type=jnp.float32)
        else:
            ha, hb = halves[k - 1]
            z = jnp.dot(ha[...], W_ref[k, :DHALF, :],
                        preferred_element_type=jnp.float32)
            z = z + jnp.dot(hb[...], W_ref[k, DHALF:, :],
                            preferred_element_type=jnp.float32)
        z = z + b_ref[k]
        z = jnp.maximum(z, 0.0)
        z = _ln(z, g_ref[k], be_ref[k])
        acc = z if acc is None else acc + z
    out = jnp.dot(acc, Wf_ref[...], preferred_element_type=jnp.float32)
    out = jnp.maximum(out + bf_ref[0], 0.0)
    o_ref[...] = _ln(out, gf_ref[0], bef_ref[0])


def _tc_mlps(xp, o1, o2, o3, W, b, g, be, Wf, bf, gf, bef):
    row_spec = pl.BlockSpec((ROW_BLK, D_IN), lambda i: (i, 0))
    half_a = pl.BlockSpec((ROW_BLK, DHALF), lambda i: (i, 0))
    half_b = pl.BlockSpec((ROW_BLK, DHALF), lambda i: (i + N_BLKS, 0))
    full = lambda *shape: pl.BlockSpec(shape, lambda i: (0,) * len(shape))
    return pl.pallas_call(
        _mlp_body,
        grid=(N_BLKS,),
        in_specs=[
            row_spec,
            half_a, half_b, half_a, half_b, half_a, half_b,
            full(N_MLP, D_IN, D_H),
            full(N_MLP, D_H), full(N_MLP, D_H), full(N_MLP, D_H),
            full(D_H, D_H),
            full(1, D_H), full(1, D_H), full(1, D_H),
        ],
        out_specs=pl.BlockSpec((ROW_BLK, D_H), lambda i: (i, 0)),
        out_shape=jax.ShapeDtypeStruct((N_SP, D_H), jnp.float32),
    )(xp, o1, o1, o2, o2, o3, o3, W, b, g, be, Wf,
      bf.reshape(1, D_H), gf.reshape(1, D_H), bef.reshape(1, D_H))


@jax.jit
def kernel(x, edge_index, W, b, g, be, Wf, bf, gf, bef):
    src, dst = edge_index[0], edge_index[1]
    pad_e = E_PAD - E

    def to_2d(v, fill):
        vp = jnp.concatenate([v, jnp.full((pad_e,), fill, jnp.int32)])
        vp = vp.reshape(NS, CHUNKS_PER_TILE, CHUNK)
        dummy = jnp.full((NS, TILE_STRIDE - CHUNKS_PER_TILE, CHUNK),
                         fill, jnp.int32)
        return jnp.concatenate([vp, dummy], axis=1).reshape(
            NS * TILE_STRIDE, CHUNK)

    src_p = to_2d(src, 0)
    dst_p = to_2d(dst, N)

    xp = jnp.pad(x, ((0, N_SP - N), (0, 0)))
    xcat = jnp.concatenate([xp[:, :DHALF], xp[:, DHALF:]])

    o1, o2, o3 = _sc_propagate(xcat, src_p, dst_p)
    out = _tc_mlps(xp, o1, o2, o3, W, b, g, be, Wf, bf, gf, bef)
    return out[:N]


# speedup vs baseline: 1.1590x; 1.0110x over previous
import jax
import jax.numpy as jnp
from jax import lax
from jax.experimental import pallas as pl
from jax.experimental.pallas import tpu as pltpu
from jax.experimental.pallas import tpu_sc as plsc

N = 10000
E = 160000
D_IN = 256
D_H = 512
N_HOPS = 3
N_MLP = N_HOPS + 1

NC, NS, LANES = 2, 16, 16
DHALF = D_IN // 2
ROWS_PER_TILE = 640
N_SP = NS * ROWS_PER_TILE
CHUNK = 128
CHUNKS_PER_TILE = 79
TILE_STRIDE = 80
E_PAD = NS * CHUNKS_PER_TILE * CHUNK
NORM_CH = 128
ROW_BLK = 256
N_BLKS = N_SP // ROW_BLK


def _sc_body(xcat_hbm, src_hbm, dst_hbm, o1_hbm, o2_hbm, o3_hbm,
             acc_sp, deg_sp, sem_a, sem_b):
    c = lax.axis_index("c")
    s = lax.axis_index("s")
    c_off = c * N_SP
    tile_row0 = s * ROWS_PER_TILE

    def scoped(src2d, dst2d, rows_a, ones_v, dinv):
        buf = rows_a
        for t in range(ROWS_PER_TILE // LANES):
            dinv[pl.ds(t * LANES, LANES)] = jnp.zeros((LANES,), jnp.float32)
        for t in range(CHUNK // LANES):
            ones_v[pl.ds(t * LANES, LANES)] = jnp.ones((LANES,), jnp.float32)
        pltpu.sync_copy(dinv, deg_sp.at[pl.ds(tile_row0, ROWS_PER_TILE)])

        pltpu.sync_copy(src_hbm.at[pl.ds(s * TILE_STRIDE, TILE_STRIDE)], src2d)
        pltpu.sync_copy(dst_hbm.at[pl.ds(s * TILE_STRIDE, TILE_STRIDE)], dst2d)

        def add_off(j, carry):
            for t in range(CHUNK // LANES):
                sl = pl.ds(t * LANES, LANES)
                src2d[j, sl] = src2d[j, sl] + c_off
            return carry
        lax.fori_loop(0, CHUNKS_PER_TILE, add_off, 0)
        plsc.subcore_barrier()

        outs = [o1_hbm, o2_hbm, o3_hbm]
        h_in = xcat_hbm
        for hop in range(N_HOPS):
            if hop == 0:
                for ch in range(ROWS_PER_TILE // NORM_CH):
                    r0 = tile_row0 + ch * NORM_CH
                    pltpu.sync_copy(h_in.at[pl.ds(c_off + r0, NORM_CH)], buf)
                    pltpu.sync_copy(buf, acc_sp.at[pl.ds(r0, NORM_CH)])
                plsc.subcore_barrier()

            def chunk_body(j, carry):
                pltpu.async_copy(h_in.at[src2d.at[j]], rows_a, sem_a).wait()
                pltpu.sync_copy(rows_a, acc_sp.at[dst2d.at[j]], add=True)
                if hop == 0:
                    pltpu.sync_copy(ones_v, deg_sp.at[dst2d.at[j]], add=True)
                return carry
            lax.fori_loop(0, CHUNKS_PER_TILE, chunk_body, 0)
            plsc.subcore_barrier()

            if hop == 0:
                pltpu.sync_copy(deg_sp.at[pl.ds(tile_row0, ROWS_PER_TILE)],
                                dinv)
                for t in range(ROWS_PER_TILE // LANES):
                    dv = dinv[pl.ds(t * LANES, LANES)]
                    dinv[pl.ds(t * LANES, LANES)] = 1.0 / (1.0 + dv)

            out = outs[hop]
            for ch in range(ROWS_PER_TILE // NORM_CH):
                row0 = tile_row0 + ch * NORM_CH
                pltpu.sync_copy(acc_sp.at[pl.ds(row0, NORM_CH)], buf)

                def norm_grp(grp, carry):
                    dvv = dinv[pl.ds(ch * NORM_CH + grp * LANES, LANES)]
                    for r in range(LANES):
                        dv = dvv[r]
                        row = grp * LANES + r
                        for t in range(DHALF // LANES):
                            sl = pl.ds(t * LANES, LANES)
                            buf[row, sl] = buf[row, sl] * dv
                    return carry
                lax.fori_loop(0, NORM_CH // LANES, norm_grp, 0)
                pltpu.sync_copy(buf, out.at[pl.ds(c_off + row0, NORM_CH)])
                if hop < N_HOPS - 1:
                    pltpu.sync_copy(buf, acc_sp.at[pl.ds(row0, NORM_CH)])
            plsc.subcore_barrier()
            h_in = out

    pl.run_scoped(scoped,
                  pltpu.VMEM((TILE_STRIDE, CHUNK), jnp.int32),
                  pltpu.VMEM((TILE_STRIDE, CHUNK), jnp.int32),
                  pltpu.VMEM((CHUNK, DHALF), jnp.float32),
                  pltpu.VMEM((CHUNK,), jnp.float32),
                  pltpu.VMEM((ROWS_PER_TILE,), jnp.float32))


def _sc_propagate(xcat, src_p, dst_p):
    mesh = plsc.VectorSubcoreMesh(core_axis_name="c", subcore_axis_name="s",
                                  num_cores=NC, num_subcores=NS)
    k = pl.kernel(
        _sc_body,
        out_type=[jax.ShapeDtypeStruct((NC * N_SP, DHALF), jnp.float32)] * 3,
        mesh=mesh,
        scratch_types=[
            pltpu.VMEM_SHARED((N_SP, DHALF), jnp.float32),
            pltpu.VMEM_SHARED((N_SP,), jnp.float32),
            pltpu.SemaphoreType.DMA,
            pltpu.SemaphoreType.DMA,
        ],
    )
    return k(xcat, src_p, dst_p)


def _ln(z, gamma, beta):
    mu = jnp.mean(z, axis=1, keepdims=True)
    d = z - mu
    var = jnp.mean(d * d, axis=1, keepdims=True)
    return d * lax.rsqrt(var + 1e-5) * gamma + beta


def _mlp_body(x_ref, a1, b1, a2, b2, a3, b3, W_ref, b_ref, g_ref, be_ref,
              Wf_ref, bf_ref, gf_ref, bef_ref, o_ref):
    halves = [(a1, b1), (a2, b2), (a3, b3)]
    acc = None
    for k in range(N_MLP):
        if k == 0:
            z = jnp.dot(x_ref[...], W_ref[0],
                        preferred_element_type=jnp.float32)
        else:
            ha, hb = halves[k - 1]
            z = jnp.dot(ha[...], W_ref[k, :DHALF, :],
                        preferred_element_type=jnp.float32)
            z = z + jnp.dot(hb[...], W_ref[k, DHALF:, :],
                            preferred_element_type=jnp.float32)
        z = z + b_ref[k]
        z = jnp.maximum(z, 0.0)
        z = _ln(z, g_ref[k], be_ref[k])
        acc = z if acc is None else acc + z
    out = jnp.dot(acc, Wf_ref[...], preferred_element_type=jnp.float32)
    out = jnp.maximum(out + bf_ref[0], 0.0)
    o_ref[...] = _ln(out, gf_ref[0], bef_ref[0])


def _tc_mlps(xp, o1, o2, o3, W, b, g, be, Wf, bf, gf, bef):
    row_spec = pl.BlockSpec((ROW_BLK, D_IN), lambda i: (i, 0))
    half_a = pl.BlockSpec((ROW_BLK, DHALF), lambda i: (i, 0))
    half_b = pl.BlockSpec((ROW_BLK, DHALF), lambda i: (i + N_BLKS, 0))
    full = lambda *shape: pl.BlockSpec(shape, lambda i: (0,) * len(shape))
    return pl.pallas_call(
        _mlp_body,
        grid=(N_BLKS,),
        in_specs=[
            row_spec,
            half_a, half_b, half_a, half_b, half_a, half_b,
            full(N_MLP, D_IN, D_H),
            full(N_MLP, D_H), full(N_MLP, D_H), full(N_MLP, D_H),
            full(D_H, D_H),
            full(1, D_H), full(1, D_H), full(1, D_H),
        ],
        out_specs=pl.BlockSpec((ROW_BLK, D_H), lambda i: (i, 0)),
        out_shape=jax.ShapeDtypeStruct((N_SP, D_H), jnp.float32),
    )(xp, o1, o1, o2, o2, o3, o3, W, b, g, be, Wf,
      bf.reshape(1, D_H), gf.reshape(1, D_H), bef.reshape(1, D_H))


@jax.jit
def kernel(x, edge_index, W, b, g, be, Wf, bf, gf, bef):
    src, dst = edge_index[0], edge_index[1]
    pad_e = E_PAD - E

    def to_2d(v, fill):
        vp = jnp.concatenate([v, jnp.full((pad_e,), fill, jnp.int32)])
        vp = vp.reshape(NS, CHUNKS_PER_TILE, CHUNK)
        dummy = jnp.full((NS, TILE_STRIDE - CHUNKS_PER_TILE, CHUNK),
                         fill, jnp.int32)
        return jnp.concatenate([vp, dummy], axis=1).reshape(
            NS * TILE_STRIDE, CHUNK)

    src_p = to_2d(src, 0)
    dst_p = to_2d(dst, N)

    xp = jnp.pad(x, ((0, N_SP - N), (0, 0)))
    xcat = jnp.concatenate([xp[:, :DHALF], xp[:, DHALF:]])

    o1, o2, o3 = _sc_propagate(xcat, src_p, dst_p)
    out = _tc_mlps(xp, o1, o2, o3, W, b, g, be, Wf, bf, gf, bef)
    return out[:N]
